# Initial kernel scaffold; baseline (speedup 1.0000x reference)
import jax
import jax.numpy as jnp
from jax.experimental import pallas as pl


def kernel(users_emb, items_emb, edge_index):
    raise NotImplementedError("write your pallas kernel here")


# speedup vs baseline: 4.9076x; 4.9076x over previous
import jax
import jax.numpy as jnp
from jax import lax
from jax.experimental import pallas as pl
from jax.experimental.pallas import tpu as pltpu
from jax.experimental.pallas import tpu_sc as plsc

NC = 2
NS = 16
NW = NC * NS
CHUNK = 80
NPAD = 10240
D = 128
BLK = 1280

_MESH = plsc.VectorSubcoreMesh(
    core_axis_name="c", subcore_axis_name="s", num_cores=NC, num_subcores=NS
)


def _deg_body(eidx, ones_hbm, zeros_hbm, deg_out, ib, ones_v, acc):
    nch = eidx.shape[1]
    c = lax.axis_index("c")
    s = lax.axis_index("s")
    wid = c * NS + s
    rpt = NPAD // NS
    pltpu.sync_copy(zeros_hbm.at[pl.ds(s * rpt, rpt)], acc.at[pl.ds(s * rpt, rpt)])
    pltpu.sync_copy(ones_hbm, ones_v)
    plsc.subcore_barrier()

    def step(g, carry):
        pltpu.sync_copy(eidx.at[wid, g], ib)
        pltpu.sync_copy(ones_v, acc.at[ib.at[0]], add=True)
        return carry

    lax.fori_loop(0, nch, step, 0)
    plsc.subcore_barrier()
    pltpu.sync_copy(acc.at[pl.ds(s * rpt, rpt)], deg_out.at[c, pl.ds(s * rpt, rpt)])


def _agg_body(eidx, h_hbm, zeros_hbm, agg_out, ib, rows, acc, sem):
    nch = eidx.shape[1]
    c = lax.axis_index("c")
    s = lax.axis_index("s")
    wid = c * NS + s
    rpt = NPAD // NS
    pltpu.sync_copy(zeros_hbm.at[pl.ds(s * rpt, rpt)], acc.at[pl.ds(s * rpt, rpt)])
    plsc.subcore_barrier()

    def step(g, carry):
        pltpu.sync_copy(eidx.at[wid, g], ib)
        pltpu.async_copy(h_hbm.at[ib.at[0]], rows, sem).wait()
        pltpu.sync_copy(rows, acc.at[ib.at[1]], add=True)
        return carry

    lax.fori_loop(0, nch, step, 0)
    plsc.subcore_barrier()
    pltpu.sync_copy(acc.at[pl.ds(s * rpt, rpt)], agg_out.at[c, pl.ds(s * rpt, rpt)])


def _scale_body(deg_ref, emb_ref, h_ref):
    deg = deg_ref[0] + deg_ref[1]
    normed = lax.rsqrt(jnp.maximum(deg, 1.0))
    h_ref[...] = emb_ref[...] * normed[:, None]


def _final_body(deg_ref, agg_ref, out_ref):
    deg = deg_ref[0] + deg_ref[1]
    normed = lax.rsqrt(jnp.maximum(deg, 1.0))
    out_ref[...] = (agg_ref[0] + agg_ref[1]) * normed[:, None]


def kernel(users_emb, items_emb, edge_index):
    n_users, d = users_emb.shape
    n_items = items_emb.shape[0]
    n = n_users + n_items
    e = edge_index.shape[1]
    assert d == D and n <= NPAD and e % (NW * CHUNK) == 0
    nch = e // (NW * CHUNK)

    ei = edge_index.astype(jnp.int32)
    eidx = ei.reshape(2, NW, nch, CHUNK).transpose(1, 2, 0, 3)
    all_emb = jnp.concatenate(
        [users_emb, items_emb, jnp.zeros((NPAD - n, D), jnp.float32)], axis=0
    )
    ones = jnp.ones((CHUNK,), jnp.float32)
    zeros1 = jnp.zeros((NPAD,), jnp.float32)
    zeros128 = jnp.zeros((NPAD, D), jnp.float32)

    deg_parts = pl.kernel(
        _deg_body,
        out_type=jax.ShapeDtypeStruct((NC, NPAD), jnp.float32),
        mesh=_MESH,
        scratch_types=[
            pltpu.VMEM((2, CHUNK), jnp.int32),
            pltpu.VMEM((CHUNK,), jnp.float32),
            pltpu.VMEM_SHARED((NPAD,), jnp.float32),
        ],
    )(eidx, ones, zeros1)

    h = pl.pallas_call(
        _scale_body,
        grid=(NPAD // BLK,),
        in_specs=[
            pl.BlockSpec((2, BLK), lambda i: (0, i)),
            pl.BlockSpec((BLK, D), lambda i: (i, 0)),
        ],
        out_specs=pl.BlockSpec((BLK, D), lambda i: (i, 0)),
        out_shape=jax.ShapeDtypeStruct((NPAD, D), jnp.float32),
    )(deg_parts, all_emb)

    agg_parts = pl.kernel(
        _agg_body,
        out_type=jax.ShapeDtypeStruct((NC, NPAD, D), jnp.float32),
        mesh=_MESH,
        scratch_types=[
            pltpu.VMEM((2, CHUNK), jnp.int32),
            pltpu.VMEM((CHUNK, D), jnp.float32),
            pltpu.VMEM_SHARED((NPAD, D), jnp.float32),
            pltpu.SemaphoreType.DMA,
        ],
    )(eidx, h, zeros128)

    out = pl.pallas_call(
        _final_body,
        grid=(NPAD // BLK,),
        in_specs=[
            pl.BlockSpec((2, BLK), lambda i: (0, i)),
            pl.BlockSpec((2, BLK, D), lambda i: (0, i, 0)),
        ],
        out_specs=pl.BlockSpec((BLK, D), lambda i: (i, 0)),
        out_shape=jax.ShapeDtypeStruct((NPAD, D), jnp.float32),
    )(deg_parts, agg_parts)

    return out[:n]
